# Initial kernel scaffold; baseline (speedup 1.0000x reference)
import functools

import jax
import jax.numpy as jnp
from jax import lax
from jax.experimental import pallas as pl
from jax.experimental.pallas import tpu as pltpu
from jax.experimental.pallas import tpu_sc as plsc

N = 16384
D_IN = 16384
D_OUT = 64
NNZ = 2621440

NW = 32
ROWS_W = N // NW
CH = 512
SUB = 128
NKV = D_OUT // 16

_mesh = plsc.VectorSubcoreMesh(core_axis_name="c", subcore_axis_name="s")


@functools.partial(
    pl.kernel,
    out_type=jax.ShapeDtypeStruct((N, D_OUT), jnp.float32),
    mesh=_mesh,
    scratch_types=[
        pltpu.VMEM((48,), jnp.int32),
        pltpu.VMEM((CH,), jnp.int32),
        pltpu.VMEM((CH,), jnp.int32),
        pltpu.VMEM((CH,), jnp.float32),
        pltpu.VMEM((CH, D_OUT), jnp.float32),
        pltpu.VMEM((ROWS_W, D_OUT), jnp.float32),
        pltpu.VMEM((D_OUT,), jnp.float32),
        pltpu.SemaphoreType.DMA,
    ],
)
def _spmm_sc(values_hbm, row_hbm, col_hbm, w_hbm, b_hbm, bounds_hbm,
             out_hbm, bounds_v, cidx_v, ridx_v, vals_v, rows_v, acc_v,
             b_v, sem):
    wid = lax.axis_index("s") * 2 + lax.axis_index("c")
    row0 = wid * ROWS_W

    pltpu.sync_copy(bounds_hbm, bounds_v)
    pltpu.sync_copy(b_hbm, b_v)

    lo = bounds_v[wid]
    hi = bounds_v[wid + 1]
    lo8 = (lo // 8) * 8
    n_ch = (hi - lo8 + CH - 1) // CH

    def init_body(r, _):
        for k in range(NKV):
            acc_v[r, pl.ds(16 * k, 16)] = b_v[pl.ds(16 * k, 16)]
        return 0
    lax.fori_loop(0, ROWS_W, init_body, 0)

    def chunk_body(i, _):
        start = lo8 + i * CH
        s = jnp.minimum(start, NNZ - CH)
        pltpu.sync_copy(values_hbm.at[pl.ds(s, CH)], vals_v)
        pltpu.sync_copy(row_hbm.at[pl.ds(s, CH)], ridx_v)
        pltpu.sync_copy(col_hbm.at[pl.ds(s, CH)], cidx_v)
        copies = []
        for q in range(CH // SUB):
            copies.append(pltpu.async_copy(
                w_hbm.at[cidx_v.at[pl.ds(q * SUB, SUB)]],
                rows_v.at[pl.ds(q * SUB, SUB)], sem))
        for c in copies:
            c.wait()

        w_lo = jnp.maximum(lo, start) - s
        w_hi = jnp.minimum(hi, start + CH) - s

        def nnz_body(j, _):
            valid = (j >= w_lo) & (j < w_hi)
            r = jnp.clip(ridx_v[j] - row0, 0, ROWS_W - 1)
            v = jnp.where(valid, vals_v[j], 0.0)
            for k in range(NKV):
                plsc.addupdate(acc_v.at[r, pl.ds(16 * k, 16)],
                               v * rows_v[j, pl.ds(16 * k, 16)])
            return 0
        lax.fori_loop(0, CH, nnz_body, 0)
        return 0

    lax.fori_loop(0, n_ch, chunk_body, 0)

    pltpu.sync_copy(acc_v, out_hbm.at[pl.ds(row0, ROWS_W)])


def kernel(values, row_idx, col_idx, W, b):
    edges = jnp.arange(0, N + 1, ROWS_W, dtype=jnp.int32)
    bounds = jnp.searchsorted(row_idx, edges).astype(jnp.int32)
    bounds = jnp.concatenate(
        [bounds, jnp.full((48 - bounds.shape[0],), NNZ, jnp.int32)])
    return _spmm_sc(values, row_idx, col_idx, W, b, bounds)


# speedup vs baseline: 8.8014x; 8.8014x over previous
import functools

import jax
import jax.numpy as jnp
from jax import lax
from jax.experimental import pallas as pl
from jax.experimental.pallas import tpu as pltpu
from jax.experimental.pallas import tpu_sc as plsc

N = 16384
D_IN = 16384
D_OUT = 64
NNZ = 2621440

NW = 32
ROWS_W = N // NW
CH = 512
SUB = 128
NKV = D_OUT // 16

_mesh = plsc.VectorSubcoreMesh(core_axis_name="c", subcore_axis_name="s")


@functools.partial(
    pl.kernel,
    out_type=jax.ShapeDtypeStruct((N, D_OUT), jnp.float32),
    mesh=_mesh,
    compiler_params=pltpu.CompilerParams(use_tc_tiling_on_sc=False),
    scratch_types=[
        pltpu.VMEM((48,), jnp.int32),
        pltpu.VMEM((CH,), jnp.int32),
        pltpu.VMEM((CH,), jnp.int32),
        pltpu.VMEM((CH,), jnp.float32),
        pltpu.VMEM((CH, D_OUT), jnp.float32),
        pltpu.VMEM((ROWS_W, D_OUT), jnp.float32),
        pltpu.VMEM((D_OUT,), jnp.float32),
        pltpu.SemaphoreType.DMA,
    ],
)
def _spmm_sc(values_hbm, row_hbm, col_hbm, w_hbm, b_hbm, bounds_hbm,
             out_hbm, bounds_v, cidx_v, ridx_v, vals_v, rows_v, acc_v,
             b_v, sem):
    wid = lax.axis_index("s") * 2 + lax.axis_index("c")
    row0 = wid * ROWS_W

    pltpu.sync_copy(bounds_hbm, bounds_v)
    pltpu.sync_copy(b_hbm, b_v)

    bv = bounds_v[pl.ds(wid, 16)]
    lo = bv[0]
    hi = bv[1]
    lo8 = (lo // 8) * 8
    n_ch = (hi - lo8 + CH - 1) // CH

    def init_body(r, _):
        for k in range(NKV):
            acc_v[r, pl.ds(16 * k, 16)] = b_v[pl.ds(16 * k, 16)]
        return 0
    lax.fori_loop(0, ROWS_W, init_body, 0)

    def chunk_body(i, _):
        start = lo8 + i * CH
        s = jnp.minimum(start, NNZ - CH)
        pltpu.sync_copy(values_hbm.at[pl.ds(s, CH)], vals_v)
        pltpu.sync_copy(row_hbm.at[pl.ds(s, CH)], ridx_v)
        pltpu.sync_copy(col_hbm.at[pl.ds(s, CH)], cidx_v)
        copies = []
        for q in range(CH // SUB):
            copies.append(pltpu.async_copy(
                w_hbm.at[cidx_v.at[pl.ds(q * SUB, SUB)]],
                rows_v.at[pl.ds(q * SUB, SUB)], sem))
        for c in copies:
            c.wait()

        w_lo = jnp.maximum(lo, start) - s
        w_hi = jnp.minimum(hi, start + CH) - s

        lane = lax.iota(jnp.int32, 16)

        def grp_body(g, _):
            j16 = g * 16
            rclip = jnp.clip(ridx_v[pl.ds(j16, 16)] - row0, 0, ROWS_W - 1)
            gidx = j16 + lane
            vmask = (gidx >= w_lo) & (gidx < w_hi)
            vval = jnp.where(vmask, vals_v[pl.ds(j16, 16)], 0.0)
            for t in range(16):
                r = rclip[t]
                v = vval[t]
                for k in range(NKV):
                    plsc.addupdate(acc_v.at[r, pl.ds(16 * k, 16)],
                                   v * rows_v[j16 + t, pl.ds(16 * k, 16)])
            return 0
        lax.fori_loop(0, CH // 16, grp_body, 0)
        return 0

    lax.fori_loop(0, n_ch, chunk_body, 0)

    pltpu.sync_copy(acc_v, out_hbm.at[pl.ds(row0, ROWS_W)])


def kernel(values, row_idx, col_idx, W, b):
    edges = jnp.arange(0, N + 1, ROWS_W, dtype=jnp.int32)
    bounds = jnp.searchsorted(row_idx, edges).astype(jnp.int32)
    bounds = jnp.concatenate(
        [bounds, jnp.full((48 - bounds.shape[0],), NNZ, jnp.int32)])
    return _spmm_sc(values, row_idx, col_idx, W, b, bounds)


# speedup vs baseline: 29.4451x; 3.3455x over previous
import functools

import jax
import jax.numpy as jnp
from jax import lax
from jax.experimental import pallas as pl
from jax.experimental.pallas import tpu as pltpu
from jax.experimental.pallas import tpu_sc as plsc

N = 16384
D_IN = 16384
D_OUT = 64
NNZ = 2621440

NW = 32
ROWS_W = N // NW
CH = 512
SUB = 128
NKV = D_OUT // 16

_mesh = plsc.VectorSubcoreMesh(core_axis_name="c", subcore_axis_name="s")


@functools.partial(
    pl.kernel,
    out_type=jax.ShapeDtypeStruct((N, D_OUT), jnp.float32),
    mesh=_mesh,
    compiler_params=pltpu.CompilerParams(use_tc_tiling_on_sc=False),
    scratch_types=[
        pltpu.VMEM((48,), jnp.int32),
        pltpu.VMEM((2, CH), jnp.int32),
        pltpu.VMEM((2, CH), jnp.int32),
        pltpu.VMEM((2, CH), jnp.float32),
        pltpu.VMEM((2, CH, D_OUT), jnp.float32),
        pltpu.VMEM((ROWS_W, D_OUT), jnp.float32),
        pltpu.VMEM((D_OUT,), jnp.float32),
        pltpu.SemaphoreType.DMA,
        pltpu.SemaphoreType.DMA,
        pltpu.SemaphoreType.DMA,
        pltpu.SemaphoreType.DMA,
    ],
)
def _spmm_sc(values_hbm, row_hbm, col_hbm, w_hbm, b_hbm, bounds_hbm,
             out_hbm, bounds_v, cidx_v, ridx_v, vals_v, rows_v, acc_v,
             b_v, sem_g0, sem_g1, sem_s0, sem_s1):
    wid = lax.axis_index("s") * 2 + lax.axis_index("c")
    row0 = wid * ROWS_W

    pltpu.sync_copy(bounds_hbm, bounds_v)
    pltpu.sync_copy(b_hbm, b_v)

    bv = bounds_v[pl.ds(wid, 16)]
    lo = bv[0]
    hi = bv[1]
    lo8 = (lo // 8) * 8
    n_ch = (hi - lo8 + CH - 1) // CH
    n_pair = (n_ch + 1) // 2

    sem_g = (sem_g0, sem_g1)
    sem_s = (sem_s0, sem_s1)

    binit = [b_v[pl.ds(16 * k, 16)] for k in range(NKV)]

    @plsc.parallel_loop(0, ROWS_W)
    def _(r):
        for k in range(NKV):
            acc_v[r, pl.ds(16 * k, 16)] = binit[k]

    def chunk_start(i):
        return jnp.minimum(lo8 + i * CH, NNZ - CH)

    def issue(i, b):
        s = chunk_start(i)
        pltpu.sync_copy(col_hbm.at[pl.ds(s, CH)], cidx_v.at[b])
        for q in range(CH // SUB):
            pltpu.async_copy(
                w_hbm.at[cidx_v.at[b, pl.ds(q * SUB, SUB)]],
                rows_v.at[b, pl.ds(q * SUB, SUB)], sem_g[b])
        pltpu.async_copy(values_hbm.at[pl.ds(s, CH)], vals_v.at[b],
                         sem_s[b])
        pltpu.async_copy(row_hbm.at[pl.ds(s, CH)], ridx_v.at[b],
                         sem_s[b])

    def drain(b):
        for q in range(CH // SUB):
            pltpu.make_async_copy(
                w_hbm.at[pl.ds(0, SUB)],
                rows_v.at[b, pl.ds(q * SUB, SUB)], sem_g[b]).wait()
        pltpu.make_async_copy(values_hbm.at[pl.ds(0, CH)], vals_v.at[b],
                              sem_s[b]).wait()
        pltpu.make_async_copy(row_hbm.at[pl.ds(0, CH)], ridx_v.at[b],
                              sem_s[b]).wait()

    lane = lax.iota(jnp.int32, 16)

    def compute(i, b):
        s = chunk_start(i)
        start = lo8 + i * CH
        w_lo = jnp.maximum(lo, start) - s
        w_hi = jnp.minimum(hi, start + CH) - s

        @plsc.parallel_loop(0, CH, step=16)
        def _(j16):
            rclip = jnp.clip(ridx_v[b, pl.ds(j16, 16)] - row0,
                             0, ROWS_W - 1)
            gidx = j16 + lane
            vmask = (gidx >= w_lo) & (gidx < w_hi)
            vval = jnp.where(vmask, vals_v[b, pl.ds(j16, 16)], 0.0)
            for t in range(16):
                r = rclip[t]
                v = vval[t]
                for k in range(NKV):
                    plsc.addupdate(
                        acc_v.at[r, pl.ds(16 * k, 16)],
                        v * rows_v[b, j16 + t, pl.ds(16 * k, 16)])

    issue(0, 0)

    def pair_body(p, _):
        issue(2 * p + 1, 1)
        drain(0)
        compute(2 * p, 0)
        issue(2 * p + 2, 0)
        drain(1)
        compute(2 * p + 1, 1)
        return 0

    lax.fori_loop(0, n_pair, pair_body, 0)
    drain(0)

    pltpu.sync_copy(acc_v, out_hbm.at[pl.ds(row0, ROWS_W)])


def kernel(values, row_idx, col_idx, W, b):
    edges = jnp.arange(0, N + 1, ROWS_W, dtype=jnp.int32)
    bounds = jnp.searchsorted(row_idx, edges).astype(jnp.int32)
    bounds = jnp.concatenate(
        [bounds, jnp.full((48 - bounds.shape[0],), NNZ, jnp.int32)])
    return _spmm_sc(values, row_idx, col_idx, W, b, bounds)


# speedup vs baseline: 37.0510x; 1.2583x over previous
import functools

import jax
import jax.numpy as jnp
from jax import lax
from jax.experimental import pallas as pl
from jax.experimental.pallas import tpu as pltpu
from jax.experimental.pallas import tpu_sc as plsc

N = 16384
D_IN = 16384
D_OUT = 64
NNZ = 2621440

NW = 32
ROWS_W = N // NW
CH = 512
CHP = CH + 16
SUB = 128
NKV = D_OUT // 16
GRP = 4

_mesh = plsc.VectorSubcoreMesh(core_axis_name="c", subcore_axis_name="s")


@functools.partial(
    pl.kernel,
    out_type=jax.ShapeDtypeStruct((N, D_OUT), jnp.float32),
    mesh=_mesh,
    compiler_params=pltpu.CompilerParams(use_tc_tiling_on_sc=False),
    scratch_types=[
        pltpu.VMEM((48,), jnp.int32),
        pltpu.VMEM((2, CH), jnp.int32),
        pltpu.VMEM((2, CHP), jnp.int32),
        pltpu.VMEM((2, CHP), jnp.float32),
        pltpu.VMEM((2, CH, D_OUT), jnp.float32),
        pltpu.VMEM((ROWS_W, D_OUT), jnp.float32),
        pltpu.VMEM((D_OUT,), jnp.float32),
        pltpu.SemaphoreType.DMA,
        pltpu.SemaphoreType.DMA,
        pltpu.SemaphoreType.DMA,
        pltpu.SemaphoreType.DMA,
        pltpu.SemaphoreType.DMA,
        pltpu.SemaphoreType.DMA,
    ],
)
def _spmm_sc(values_hbm, row_hbm, col_hbm, w_hbm, b_hbm, bounds_hbm,
             out_hbm, bounds_v, cidx_v, ridx_v, vals_v, rows_v, acc_v,
             b_v, sem_g0, sem_g1, sem_s0, sem_s1, sem_c0, sem_c1):
    wid = lax.axis_index("s") * 2 + lax.axis_index("c")
    row0 = wid * ROWS_W

    pltpu.sync_copy(bounds_hbm, bounds_v)
    pltpu.sync_copy(b_hbm, b_v)

    bv = bounds_v[pl.ds(wid, 16)]
    lo = bv[0]
    hi = bv[1]
    lo8 = (lo // 8) * 8
    n_ch = (hi - lo8 + CH - 1) // CH
    n_pair = (n_ch + 1) // 2

    sem_g = (sem_g0, sem_g1)
    sem_s = (sem_s0, sem_s1)
    sem_c = (sem_c0, sem_c1)

    binit = [b_v[pl.ds(16 * k, 16)] for k in range(NKV)]

    @plsc.parallel_loop(0, ROWS_W)
    def _(r):
        for k in range(NKV):
            acc_v[r, pl.ds(16 * k, 16)] = binit[k]

    def chunk_start(i):
        return jnp.minimum(lo8 + i * CH, NNZ - CH)

    def issue_cidx(i, b):
        pltpu.async_copy(col_hbm.at[pl.ds(chunk_start(i), CH)],
                         cidx_v.at[b], sem_c[b])

    def wait_cidx(b):
        pltpu.make_async_copy(col_hbm.at[pl.ds(0, CH)], cidx_v.at[b],
                              sem_c[b]).wait()

    def issue_rest(i, b):
        s = chunk_start(i)
        for q in range(CH // SUB):
            pltpu.async_copy(
                w_hbm.at[cidx_v.at[b, pl.ds(q * SUB, SUB)]],
                rows_v.at[b, pl.ds(q * SUB, SUB)], sem_g[b])
        pltpu.async_copy(values_hbm.at[pl.ds(s, CH)],
                         vals_v.at[b, pl.ds(0, CH)], sem_s[b])
        pltpu.async_copy(row_hbm.at[pl.ds(s, CH)],
                         ridx_v.at[b, pl.ds(0, CH)], sem_s[b])

    def drain(b):
        for q in range(CH // SUB):
            pltpu.make_async_copy(
                w_hbm.at[pl.ds(0, SUB)],
                rows_v.at[b, pl.ds(q * SUB, SUB)], sem_g[b]).wait()
        pltpu.make_async_copy(values_hbm.at[pl.ds(0, CH)],
                              vals_v.at[b, pl.ds(0, CH)], sem_s[b]).wait()
        pltpu.make_async_copy(row_hbm.at[pl.ds(0, CH)],
                              ridx_v.at[b, pl.ds(0, CH)], sem_s[b]).wait()

    lane = lax.iota(jnp.int32, 16)

    def compute(i, b):
        s = chunk_start(i)
        start = lo8 + i * CH
        w_lo = jnp.maximum(lo, start) - s
        w_hi = jnp.minimum(hi, start + CH) - s

        @plsc.parallel_loop(0, CH, step=GRP)
        def _(j0):
            rclip = jnp.clip(ridx_v[b, pl.ds(j0, 16)] - row0,
                             0, ROWS_W - 1)
            gidx = j0 + lane
            vmask = (gidx >= w_lo) & (gidx < w_hi)
            vval = jnp.where(vmask, vals_v[b, pl.ds(j0, 16)], 0.0)
            for t in range(GRP):
                r = rclip[t]
                v = vval[t]
                for k in range(NKV):
                    plsc.addupdate(
                        acc_v.at[r, pl.ds(16 * k, 16)],
                        v * rows_v[b, j0 + t, pl.ds(16 * k, 16)])

    def step(i, b):
        nb = 1 - b
        wait_cidx(nb)
        issue_rest(i + 1, nb)
        drain(b)
        issue_cidx(i + 2, b)
        compute(i, b)

    pltpu.sync_copy(col_hbm.at[pl.ds(chunk_start(0), CH)], cidx_v.at[0])
    issue_rest(0, 0)
    issue_cidx(1, 1)

    def pair_body(p, _):
        step(2 * p, 0)
        step(2 * p + 1, 1)
        return 0

    lax.fori_loop(0, n_pair, pair_body, 0)

    wait_cidx(1)
    drain(0)

    pltpu.sync_copy(acc_v, out_hbm.at[pl.ds(row0, ROWS_W)])


def kernel(values, row_idx, col_idx, W, b):
    edges = jnp.arange(0, N + 1, ROWS_W, dtype=jnp.int32)
    bounds = jnp.searchsorted(row_idx, edges).astype(jnp.int32)
    bounds = jnp.concatenate(
        [bounds, jnp.full((48 - bounds.shape[0],), NNZ, jnp.int32)])
    return _spmm_sc(values, row_idx, col_idx, W, b, bounds)
